# baseline (speedup 1.0000x reference)
import jax
import jax.numpy as jnp
from jax import lax
from jax.experimental import pallas as pl
from jax.experimental.pallas import tpu as pltpu
from jax.experimental.pallas import tpu_sc as plsc

_B = 4096
_F = 32
_NC, _NS, _NL = 2, 16, 16
_NW = _NC * _NS
_BPW = _B // _NW
_CH = 1
_NCHUNK = _BPW // _CH
_CPW = _NL // _CH
_TB = _CH * 128 + 1


def _mf_body(users_hbm, items_hbm, uft_hbm, ift_hbm, ubf_hbm, ibf_hbm,
             out_hbm, uidx_v, iidx_v, tbu0_v, tbu1_v, tbu2_v, tbu3_v,
             tbu4_v, tbu5_v, tbi0_v, tbi1_v, tbi2_v, tbi3_v, tbi4_v,
             tbi5_v, ufr_v, ifr_v, ubr_v, ibr_v, out_v, sem0, sem1, sem2,
             sem3, sem4, sem5, bsem):
    wid = lax.axis_index("s") * _NC + lax.axis_index("c")
    base = wid * _BPW

    pltpu.sync_copy(users_hbm.at[pl.ds(base, _BPW)], uidx_v)
    pltpu.sync_copy(items_hbm.at[pl.ds(base, _BPW)], iidx_v)

    bias_cps = [
        pltpu.async_copy(ubf_hbm.at[0].at[uidx_v], ubr_v, bsem),
        pltpu.async_copy(ibf_hbm.at[0].at[iidx_v], ibr_v, bsem),
    ]

    tbus = (tbu0_v, tbu1_v, tbu2_v, tbu3_v, tbu4_v, tbu5_v)
    tbis = (tbi0_v, tbi1_v, tbi2_v, tbi3_v, tbi4_v, tbi5_v)
    sems = (sem0, sem1, sem2, sem3, sem4, sem5)
    lane = lax.iota(jnp.int32, _NL)

    def enqueue(k):
        p = k % 6
        uvec = uidx_v[pl.ds((k // _CPW) * _NL, _NL)]
        ivec = iidx_v[pl.ds((k // _CPW) * _NL, _NL)]
        for l in range(_CH):
            u = uvec[(k % _CPW) * _CH + l]
            i = ivec[(k % _CPW) * _CH + l]
            qu = pl.multiple_of(
                lax.shift_left(lax.shift_right_logical(u, 7), 7), 128)
            qi = pl.multiple_of(
                lax.shift_left(lax.shift_right_logical(i, 7), 7), 128)
            pltpu.async_copy(uft_hbm.at[:, pl.ds(qu, 128)],
                             tbus[p].at[:, pl.ds(l * 128, 128)], sems[p])
            pltpu.async_copy(ift_hbm.at[:, pl.ds(qi, 128)],
                             tbis[p].at[:, pl.ds(l * 128, 128)], sems[p])

    def drain_and_extract(k):
        p = k % 6
        pltpu.make_async_copy(uft_hbm.at[:, pl.ds(0, _CH * 128)],
                              tbus[p].at[:, pl.ds(0, _CH * 128)],
                              sems[p]).wait()
        pltpu.make_async_copy(ift_hbm.at[:, pl.ds(0, _CH * 128)],
                              tbis[p].at[:, pl.ds(0, _CH * 128)],
                              sems[p]).wait()
        uvec = uidx_v[pl.ds((k // _CPW) * _NL, _NL)]
        ivec = iidx_v[pl.ds((k // _CPW) * _NL, _NL)]
        for l in range(_CH):
            b = k * _CH + l
            u = uvec[(k % _CPW) * _CH + l]
            i = ivec[(k % _CPW) * _CH + l]
            cu = jnp.full((_NL,), l * 128, jnp.int32) + lax.bitwise_and(
                u, 127)
            ci = jnp.full((_NL,), l * 128, jnp.int32) + lax.bitwise_and(
                i, 127)
            lo_u = plsc.load_gather(tbus[p], [lane, cu])
            hi_u = plsc.load_gather(tbus[p], [lane + _NL, cu])
            lo_i = plsc.load_gather(tbis[p], [lane, ci])
            hi_i = plsc.load_gather(tbis[p], [lane + _NL, ci])
            ufr_v[pl.ds(b * _F, _NL)] = lo_u
            ufr_v[pl.ds(b * _F + _NL, _NL)] = hi_u
            ifr_v[pl.ds(b * _F, _NL)] = lo_i
            ifr_v[pl.ds(b * _F + _NL, _NL)] = hi_i

    for k in range(5):
        enqueue(k)
    for k in range(5, _NCHUNK):
        enqueue(k)
        drain_and_extract(k - 5)
    for k in range(_NCHUNK - 5, _NCHUNK):
        drain_and_extract(k)

    for cp in bias_cps:
        cp.wait()

    for g in range(_BPW // _NL):
        s = pl.ds(g * _NL, _NL)
        flat_row = (lane + g * _NL) * _F
        acc = ubr_v[s] + ibr_v[s] + 3.5
        for f in range(_F):
            idx = flat_row + lax.bitwise_and(lane + f, _F - 1)
            u = plsc.load_gather(ufr_v, [idx])
            v = plsc.load_gather(ifr_v, [idx])
            acc = acc + u * v
        out_v[s] = acc

    pltpu.sync_copy(out_v, out_hbm.at[pl.ds(base, _BPW)])


@jax.jit
def _mf(users, items, user_factors, item_factors, user_biases, item_biases):
    run = pl.kernel(
        _mf_body,
        out_type=jax.ShapeDtypeStruct((_B,), jnp.float32),
        mesh=plsc.VectorSubcoreMesh(core_axis_name="c", subcore_axis_name="s"),
        compiler_params=pltpu.CompilerParams(needs_layout_passes=False),
        scratch_types=[
            pltpu.VMEM((_BPW,), jnp.int32),
            pltpu.VMEM((_BPW,), jnp.int32),
            pltpu.VMEM((_F, _TB), jnp.float32),
            pltpu.VMEM((_F, _TB), jnp.float32),
            pltpu.VMEM((_F, _TB), jnp.float32),
            pltpu.VMEM((_F, _TB), jnp.float32),
            pltpu.VMEM((_F, _TB), jnp.float32),
            pltpu.VMEM((_F, _TB), jnp.float32),
            pltpu.VMEM((_F, _TB), jnp.float32),
            pltpu.VMEM((_F, _TB), jnp.float32),
            pltpu.VMEM((_F, _TB), jnp.float32),
            pltpu.VMEM((_F, _TB), jnp.float32),
            pltpu.VMEM((_F, _TB), jnp.float32),
            pltpu.VMEM((_F, _TB), jnp.float32),
            pltpu.VMEM((_BPW * _F,), jnp.float32),
            pltpu.VMEM((_BPW * _F,), jnp.float32),
            pltpu.VMEM((_BPW,), jnp.float32),
            pltpu.VMEM((_BPW,), jnp.float32),
            pltpu.VMEM((_BPW,), jnp.float32),
            pltpu.SemaphoreType.DMA,
            pltpu.SemaphoreType.DMA,
            pltpu.SemaphoreType.DMA,
            pltpu.SemaphoreType.DMA,
            pltpu.SemaphoreType.DMA,
            pltpu.SemaphoreType.DMA,
            pltpu.SemaphoreType.DMA,
        ],
    )
    return run(users, items, user_factors.T, item_factors.T,
               user_biases.T, item_biases.T)


def kernel(users, items, user_factors, item_factors, user_biases, item_biases):
    return _mf(users, items, user_factors, item_factors, user_biases,
               item_biases)


# speedup vs baseline: 1.0141x; 1.0141x over previous
import jax
import jax.numpy as jnp
from jax import lax
from jax.experimental import pallas as pl
from jax.experimental.pallas import tpu as pltpu
from jax.experimental.pallas import tpu_sc as plsc

_B = 4096
_F = 32
_NC, _NS, _NL = 2, 16, 16
_NW = _NC * _NS
_BPW = _B // _NW
_CH = 2
_NCHUNK = _BPW // _CH
_CPW = _NL // _CH
_TB = _CH * 128 + 1


def _mf_body(users_hbm, items_hbm, uft_hbm, ift_hbm, ubf_hbm, ibf_hbm,
             out_hbm, uidx_v, iidx_v, tbu0_v, tbu1_v, tbu2_v, tbu3_v,
             tbi0_v, tbi1_v, tbi2_v, tbi3_v, ufr_v, ifr_v, ubr_v, ibr_v,
             out_v, sem0, sem1, sem2, sem3, bsem):
    wid = lax.axis_index("s") * _NC + lax.axis_index("c")
    base = wid * _BPW

    pltpu.sync_copy(users_hbm.at[pl.ds(base, _BPW)], uidx_v)
    pltpu.sync_copy(items_hbm.at[pl.ds(base, _BPW)], iidx_v)

    bias_cps = [
        pltpu.async_copy(ubf_hbm.at[0].at[uidx_v], ubr_v, bsem),
        pltpu.async_copy(ibf_hbm.at[0].at[iidx_v], ibr_v, bsem),
    ]

    tbus = (tbu0_v, tbu1_v, tbu2_v, tbu3_v)
    tbis = (tbi0_v, tbi1_v, tbi2_v, tbi3_v)
    sems = (sem0, sem1, sem2, sem3)
    lane = lax.iota(jnp.int32, _NL)

    def enqueue(k):
        p = k % 4
        uvec = uidx_v[pl.ds((k // _CPW) * _NL, _NL)]
        ivec = iidx_v[pl.ds((k // _CPW) * _NL, _NL)]
        for l in range(_CH):
            u = uvec[(k % _CPW) * _CH + l]
            i = ivec[(k % _CPW) * _CH + l]
            qu = pl.multiple_of(
                lax.shift_left(lax.shift_right_logical(u, 7), 7), 128)
            qi = pl.multiple_of(
                lax.shift_left(lax.shift_right_logical(i, 7), 7), 128)
            pltpu.async_copy(uft_hbm.at[:, pl.ds(qu, 128)],
                             tbus[p].at[:, pl.ds(l * 128, 128)], sems[p])
            pltpu.async_copy(ift_hbm.at[:, pl.ds(qi, 128)],
                             tbis[p].at[:, pl.ds(l * 128, 128)], sems[p])

    def drain_and_extract(k):
        p = k % 4
        pltpu.make_async_copy(uft_hbm.at[:, pl.ds(0, _CH * 128)],
                              tbus[p].at[:, pl.ds(0, _CH * 128)],
                              sems[p]).wait()
        pltpu.make_async_copy(ift_hbm.at[:, pl.ds(0, _CH * 128)],
                              tbis[p].at[:, pl.ds(0, _CH * 128)],
                              sems[p]).wait()
        uvec = uidx_v[pl.ds((k // _CPW) * _NL, _NL)]
        ivec = iidx_v[pl.ds((k // _CPW) * _NL, _NL)]
        for l in range(_CH):
            b = k * _CH + l
            u = uvec[(k % _CPW) * _CH + l]
            i = ivec[(k % _CPW) * _CH + l]
            cu = jnp.full((_NL,), l * 128, jnp.int32) + lax.bitwise_and(
                u, 127)
            ci = jnp.full((_NL,), l * 128, jnp.int32) + lax.bitwise_and(
                i, 127)
            lo_u = plsc.load_gather(tbus[p], [lane, cu])
            hi_u = plsc.load_gather(tbus[p], [lane + _NL, cu])
            lo_i = plsc.load_gather(tbis[p], [lane, ci])
            hi_i = plsc.load_gather(tbis[p], [lane + _NL, ci])
            ufr_v[pl.ds(b * _F, _NL)] = lo_u
            ufr_v[pl.ds(b * _F + _NL, _NL)] = hi_u
            ifr_v[pl.ds(b * _F, _NL)] = lo_i
            ifr_v[pl.ds(b * _F + _NL, _NL)] = hi_i

    enqueue(0)
    enqueue(1)
    enqueue(2)
    for k in range(3, _NCHUNK):
        enqueue(k)
        drain_and_extract(k - 3)
    drain_and_extract(_NCHUNK - 3)
    drain_and_extract(_NCHUNK - 2)
    drain_and_extract(_NCHUNK - 1)

    for cp in bias_cps:
        cp.wait()

    for g in range(_BPW // _NL):
        s = pl.ds(g * _NL, _NL)
        flat_row = (lane + g * _NL) * _F
        acc = ubr_v[s] + ibr_v[s] + 3.5
        for f in range(_F):
            idx = flat_row + lax.bitwise_and(lane + f, _F - 1)
            u = plsc.load_gather(ufr_v, [idx])
            v = plsc.load_gather(ifr_v, [idx])
            acc = acc + u * v
        out_v[s] = acc

    pltpu.sync_copy(out_v, out_hbm.at[pl.ds(base, _BPW)])


@jax.jit
def _mf(users, items, user_factors, item_factors, user_biases, item_biases):
    run = pl.kernel(
        _mf_body,
        out_type=jax.ShapeDtypeStruct((_B,), jnp.float32),
        mesh=plsc.VectorSubcoreMesh(core_axis_name="c", subcore_axis_name="s"),
        compiler_params=pltpu.CompilerParams(needs_layout_passes=False),
        scratch_types=[
            pltpu.VMEM((_BPW,), jnp.int32),
            pltpu.VMEM((_BPW,), jnp.int32),
            pltpu.VMEM((_F, _TB), jnp.float32),
            pltpu.VMEM((_F, _TB), jnp.float32),
            pltpu.VMEM((_F, _TB), jnp.float32),
            pltpu.VMEM((_F, _TB), jnp.float32),
            pltpu.VMEM((_F, _TB), jnp.float32),
            pltpu.VMEM((_F, _TB), jnp.float32),
            pltpu.VMEM((_F, _TB), jnp.float32),
            pltpu.VMEM((_F, _TB), jnp.float32),
            pltpu.VMEM((_BPW * _F,), jnp.float32),
            pltpu.VMEM((_BPW * _F,), jnp.float32),
            pltpu.VMEM((_BPW,), jnp.float32),
            pltpu.VMEM((_BPW,), jnp.float32),
            pltpu.VMEM((_BPW,), jnp.float32),
            pltpu.SemaphoreType.DMA,
            pltpu.SemaphoreType.DMA,
            pltpu.SemaphoreType.DMA,
            pltpu.SemaphoreType.DMA,
            pltpu.SemaphoreType.DMA,
        ],
    )
    return run(users, items, user_factors.T, item_factors.T,
               user_biases.T, item_biases.T)


def kernel(users, items, user_factors, item_factors, user_biases, item_biases):
    return _mf(users, items, user_factors, item_factors, user_biases,
               item_biases)
